# baseline (speedup 1.0000x reference)
import functools

import jax
import jax.numpy as jnp
from jax import lax
from jax.experimental import pallas as pl
from jax.experimental.pallas import tpu as pltpu
from jax.experimental.pallas import tpu_sc as plsc

N = 10000
E = 320000
IN_CH = 128
HID = 256
HALF = HID // 2
N_CLASSES = 2

NC = 2
NS = 16
NP = 10240
RPT = NP // NS

CHUNK = 128
E_CHUNKS = E // CHUNK
FULL_PER_TILE = E_CHUNKS // NS
TAIL_CHUNKS = E_CHUNKS - FULL_PER_TILE * NS

DEG_W = 128
DEG_OW = 128
NT = NC * NS
DEG_FULL = E_CHUNKS // NT
DEG_TAIL = E_CHUNKS - DEG_FULL * NT


def _deg_body(col_ref, ones_ref, zeros_ref, out_ref,
              idx0, idx1, ones_v, sem0, sem1, dsem0, dsem1, degs):
    c = lax.axis_index("c")
    s = lax.axis_index("s")
    tile = c * NS + s
    pltpu.sync_copy(zeros_ref, degs.at[pl.ds(s * RPT, RPT)])
    pltpu.sync_copy(ones_ref, ones_v)
    plsc.subcore_barrier()

    idx = (idx0, idx1)
    sem = (sem0, sem1)
    dsem = (dsem0, dsem1)

    def start_idx(b, g):
        base = pl.multiple_of((tile + NT * g) * CHUNK, 8)
        pltpu.async_copy(col_ref.at[pl.ds(base, CHUNK)], idx[b], sem[b])

    def scatter(b):
        pltpu.make_async_copy(col_ref.at[pl.ds(0, CHUNK)], idx[b], sem[b]).wait()
        pltpu.async_copy(ones_v, degs.at[idx[b]], dsem[b], add=True)

    def drain(b):
        pltpu.make_async_copy(ones_v, degs.at[idx[b]], dsem[b]).wait()

    start_idx(0, 0)
    start_idx(1, 1)

    def body(j, carry):
        g0 = 2 * j
        scatter(0)

        @pl.when(g0 + 2 < DEG_FULL)
        def _():
            drain(0)
            start_idx(0, g0 + 2)

        scatter(1)

        @pl.when(g0 + 3 < DEG_FULL)
        def _():
            drain(1)
            start_idx(1, g0 + 3)

        return carry

    lax.fori_loop(0, DEG_FULL // 2, body, 0)
    drain(0)
    drain(1)

    @pl.when(tile < DEG_TAIL)
    def _():
        base = pl.multiple_of((E_CHUNKS - DEG_TAIL + tile) * CHUNK, 8)
        pltpu.sync_copy(col_ref.at[pl.ds(base, CHUNK)], idx[0])
        pltpu.sync_copy(ones_v, degs.at[idx[0]], add=True)

    plsc.subcore_barrier()
    pltpu.sync_copy(degs.at[pl.ds(s * RPT, RPT)], out_ref.at[c, pl.ds(s * RPT, RPT)])


_deg_kernel = functools.partial(
    pl.kernel,
    out_type=jax.ShapeDtypeStruct((NC, NP, DEG_OW), jnp.float32),
    mesh=plsc.VectorSubcoreMesh(core_axis_name="c", subcore_axis_name="s"),
    scratch_types=[
        pltpu.VMEM((CHUNK,), jnp.int32),
        pltpu.VMEM((CHUNK,), jnp.int32),
        pltpu.VMEM((CHUNK, DEG_W), jnp.float32),
        pltpu.SemaphoreType.DMA,
        pltpu.SemaphoreType.DMA,
        pltpu.SemaphoreType.DMA,
        pltpu.SemaphoreType.DMA,
        pltpu.VMEM_SHARED((NP, DEG_W), jnp.float32),
    ],
)(_deg_body)


def _msg_body(hp_ref, row_ref, col_ref, zeros_ref, out_ref,
              idx_r0, idx_c0, idx_r1, idx_c1, buf0, buf1,
              gsem0, gsem1, ssem0, ssem1, accs):
    c = lax.axis_index("c")
    s = lax.axis_index("s")
    pltpu.sync_copy(zeros_ref, accs.at[pl.ds(s * RPT, RPT)])
    plsc.subcore_barrier()

    idx_r = (idx_r0, idx_r1)
    idx_c = (idx_c0, idx_c1)
    buf = (buf0, buf1)
    gsem = (gsem0, gsem1)
    ssem = (ssem0, ssem1)

    def start_gather(b, g):
        base = pl.multiple_of((s + NS * g) * CHUNK, 8)
        pltpu.sync_copy(row_ref.at[pl.ds(base, CHUNK)], idx_r[b])
        pltpu.sync_copy(col_ref.at[pl.ds(base, CHUNK)], idx_c[b])
        pltpu.async_copy(hp_ref.at[c].at[idx_r[b]], buf[b], gsem[b])

    def fire_scatter(b):
        pltpu.make_async_copy(hp_ref.at[c].at[idx_r[b]], buf[b], gsem[b]).wait()
        pltpu.async_copy(buf[b], accs.at[idx_c[b]], ssem[b], add=True)

    def drain_scatter(b):
        pltpu.make_async_copy(buf[b], accs.at[idx_c[b]], ssem[b]).wait()

    NBUF = 2
    for b in range(NBUF):
        start_gather(b, b)

    def body(j, carry):
        g0 = NBUF * j
        for b in range(NBUF):
            fire_scatter(b)

            @pl.when(g0 + b + NBUF < FULL_PER_TILE)
            def _(b=b):
                drain_scatter(b)
                start_gather(b, g0 + b + NBUF)

        return carry

    lax.fori_loop(0, FULL_PER_TILE // NBUF, body, 0)
    for b in range(NBUF):
        drain_scatter(b)

    @pl.when(s < TAIL_CHUNKS)
    def _():
        base = pl.multiple_of((E_CHUNKS - TAIL_CHUNKS + s) * CHUNK, 8)
        pltpu.sync_copy(row_ref.at[pl.ds(base, CHUNK)], idx_r[0])
        pltpu.sync_copy(col_ref.at[pl.ds(base, CHUNK)], idx_c[0])
        pltpu.sync_copy(hp_ref.at[c].at[idx_r[0]], buf[0])
        pltpu.sync_copy(buf[0], accs.at[idx_c[0]], add=True)

    plsc.subcore_barrier()
    pltpu.sync_copy(accs.at[pl.ds(s * RPT, RPT)], out_ref.at[c, pl.ds(s * RPT, RPT)])


_msg_kernel = functools.partial(
    pl.kernel,
    out_type=jax.ShapeDtypeStruct((NC, NP, HALF), jnp.float32),
    mesh=plsc.VectorSubcoreMesh(core_axis_name="c", subcore_axis_name="s"),
    scratch_types=[
        pltpu.VMEM((CHUNK,), jnp.int32),
        pltpu.VMEM((CHUNK,), jnp.int32),
        pltpu.VMEM((CHUNK,), jnp.int32),
        pltpu.VMEM((CHUNK,), jnp.int32),
        pltpu.VMEM((CHUNK, HALF), jnp.float32),
        pltpu.VMEM((CHUNK, HALF), jnp.float32),
        pltpu.SemaphoreType.DMA,
        pltpu.SemaphoreType.DMA,
        pltpu.SemaphoreType.DMA,
        pltpu.SemaphoreType.DMA,
        pltpu.VMEM_SHARED((NP, HALF), jnp.float32),
    ],
)(_msg_body)

BLK = 400
GRID = N // BLK


def _dinv_of(degp_ref):
    deg = degp_ref[0, :, 0] + degp_ref[1, :, 0] + 1.0
    return lax.rsqrt(deg)


def _l1_body(x_ref, w_ref, degp_ref, out_ref):
    dinv = _dinv_of(degp_ref)
    h = jnp.dot(x_ref[...], w_ref[...], preferred_element_type=jnp.float32)
    hp = h * dinv[:, None]
    out_ref[0] = hp[:, :HALF]
    out_ref[1] = hp[:, HALF:]


_l1 = pl.pallas_call(
    _l1_body,
    grid=(GRID,),
    in_specs=[
        pl.BlockSpec((BLK, IN_CH), lambda i: (i, 0)),
        pl.BlockSpec((IN_CH, HID), lambda i: (0, 0)),
        pl.BlockSpec((NC, BLK, 1), lambda i: (0, i, 0)),
    ],
    out_specs=pl.BlockSpec((NC, BLK, HALF), lambda i: (0, i, 0)),
    out_shape=jax.ShapeDtypeStruct((NC, NP, HALF), jnp.float32),
)


def _mid_body(acc_ref, hp_ref, degp_ref, w_ref, b_ref, out_ref):
    dinv = _dinv_of(degp_ref)
    t = jnp.concatenate([acc_ref[0] + hp_ref[0], acc_ref[1] + hp_ref[1]], axis=1)
    z = jnp.maximum(t * dinv[:, None] + b_ref[...], 0.0)
    h = jnp.dot(z, w_ref[...], preferred_element_type=jnp.float32)
    hp = h * dinv[:, None]
    out_ref[0] = hp[:, :HALF]
    out_ref[1] = hp[:, HALF:]


_mid = pl.pallas_call(
    _mid_body,
    grid=(GRID,),
    in_specs=[
        pl.BlockSpec((NC, BLK, HALF), lambda i: (0, i, 0)),
        pl.BlockSpec((NC, BLK, HALF), lambda i: (0, i, 0)),
        pl.BlockSpec((NC, BLK, 1), lambda i: (0, i, 0)),
        pl.BlockSpec((HID, HID), lambda i: (0, 0)),
        pl.BlockSpec((HID,), lambda i: (0,)),
    ],
    out_specs=pl.BlockSpec((NC, BLK, HALF), lambda i: (0, i, 0)),
    out_shape=jax.ShapeDtypeStruct((NC, NP, HALF), jnp.float32),
)


def _out_body(acc_ref, hp_ref, degp_ref, wc_ref, b_ref, bc_ref, out_ref):
    dinv = _dinv_of(degp_ref)
    t = jnp.concatenate([acc_ref[0] + hp_ref[0], acc_ref[1] + hp_ref[1]], axis=1)
    z = jnp.maximum(t * dinv[:, None] + b_ref[...], 0.0)
    out_ref[...] = jnp.dot(z, wc_ref[...], preferred_element_type=jnp.float32) + bc_ref[...]


_out = pl.pallas_call(
    _out_body,
    grid=(GRID,),
    in_specs=[
        pl.BlockSpec((NC, BLK, HALF), lambda i: (0, i, 0)),
        pl.BlockSpec((NC, BLK, HALF), lambda i: (0, i, 0)),
        pl.BlockSpec((NC, BLK, 1), lambda i: (0, i, 0)),
        pl.BlockSpec((HID, N_CLASSES), lambda i: (0, 0)),
        pl.BlockSpec((HID,), lambda i: (0,)),
        pl.BlockSpec((N_CLASSES,), lambda i: (0,)),
    ],
    out_specs=pl.BlockSpec((BLK, N_CLASSES), lambda i: (i, 0)),
    out_shape=jax.ShapeDtypeStruct((N, N_CLASSES), jnp.float32),
)


def kernel(x, edge_index, W1, b1, W2, b2, Wc, bc):
    row = edge_index[0].astype(jnp.int32)
    col = edge_index[1].astype(jnp.int32)
    ones16 = jnp.ones((CHUNK, DEG_W), jnp.float32)
    zeros16 = jnp.zeros((RPT, DEG_W), jnp.float32)
    zeros128 = jnp.zeros((RPT, HALF), jnp.float32)

    degp = _deg_kernel(col, ones16, zeros16)
    degp = degp[:, :, :1]
    hp1 = _l1(x, W1, degp)
    acc1 = _msg_kernel(hp1, row, col, zeros128)
    hp2 = _mid(acc1, hp1, degp, W2, b1)
    acc2 = _msg_kernel(hp2, row, col, zeros128)
    return _out(acc2, hp2, degp, Wc, b2, bc)


# speedup vs baseline: 1.2932x; 1.2932x over previous
import functools

import jax
import jax.numpy as jnp
from jax import lax
from jax.experimental import pallas as pl
from jax.experimental.pallas import tpu as pltpu
from jax.experimental.pallas import tpu_sc as plsc

N = 10000
E = 320000
IN_CH = 128
HID = 256
HALF = HID // 2
N_CLASSES = 2

NC = 2
NS = 16
NP = 10240
RPT = NP // NS

CHUNK = 128
E_CHUNKS = E // CHUNK
FULL_PER_TILE = E_CHUNKS // NS
TAIL_CHUNKS = E_CHUNKS - FULL_PER_TILE * NS

DEG_W = 128
DEG_OW = 128
NT = NC * NS
DEG_FULL = E_CHUNKS // NT
DEG_TAIL = E_CHUNKS - DEG_FULL * NT


def _deg_body(col_ref, ones_ref, zeros_ref, out_ref,
              idx0, idx1, ones_v, sem0, sem1, dsem0, dsem1, degs):
    c = lax.axis_index("c")
    s = lax.axis_index("s")
    tile = c * NS + s
    pltpu.sync_copy(zeros_ref, degs.at[pl.ds(s * RPT, RPT)])
    pltpu.sync_copy(ones_ref, ones_v)
    plsc.subcore_barrier()

    idx = (idx0, idx1)
    sem = (sem0, sem1)
    dsem = (dsem0, dsem1)

    def start_idx(b, g):
        base = pl.multiple_of((tile + NT * g) * CHUNK, 8)
        pltpu.async_copy(col_ref.at[pl.ds(base, CHUNK)], idx[b], sem[b])

    def scatter(b):
        pltpu.make_async_copy(col_ref.at[pl.ds(0, CHUNK)], idx[b], sem[b]).wait()
        pltpu.async_copy(ones_v, degs.at[idx[b]], dsem[b], add=True)

    def drain(b):
        pltpu.make_async_copy(ones_v, degs.at[idx[b]], dsem[b]).wait()

    start_idx(0, 0)
    start_idx(1, 1)

    def body(j, carry):
        g0 = 2 * j
        scatter(0)

        @pl.when(g0 + 2 < DEG_FULL)
        def _():
            drain(0)
            start_idx(0, g0 + 2)

        scatter(1)

        @pl.when(g0 + 3 < DEG_FULL)
        def _():
            drain(1)
            start_idx(1, g0 + 3)

        return carry

    lax.fori_loop(0, DEG_FULL // 2, body, 0)
    drain(0)
    drain(1)

    @pl.when(tile < DEG_TAIL)
    def _():
        base = pl.multiple_of((E_CHUNKS - DEG_TAIL + tile) * CHUNK, 8)
        pltpu.sync_copy(col_ref.at[pl.ds(base, CHUNK)], idx[0])
        pltpu.sync_copy(ones_v, degs.at[idx[0]], add=True)

    plsc.subcore_barrier()
    pltpu.sync_copy(degs.at[pl.ds(s * RPT, RPT)], out_ref.at[c, pl.ds(s * RPT, RPT)])


_deg_kernel = functools.partial(
    pl.kernel,
    out_type=jax.ShapeDtypeStruct((NC, NP, DEG_OW), jnp.float32),
    mesh=plsc.VectorSubcoreMesh(core_axis_name="c", subcore_axis_name="s"),
    scratch_types=[
        pltpu.VMEM((CHUNK,), jnp.int32),
        pltpu.VMEM((CHUNK,), jnp.int32),
        pltpu.VMEM((CHUNK, DEG_W), jnp.float32),
        pltpu.SemaphoreType.DMA,
        pltpu.SemaphoreType.DMA,
        pltpu.SemaphoreType.DMA,
        pltpu.SemaphoreType.DMA,
        pltpu.VMEM_SHARED((NP, DEG_W), jnp.float32),
    ],
)(_deg_body)


def _msg_body(hp_ref, row_ref, col_ref, zeros_ref, out_ref,
              idx_r0, idx_c0, idx_r1, idx_c1, idx_r2, idx_c2, idx_r3, idx_c3,
              buf0, buf1,
              isem0, isem1, isem2, isem3, gsem0, gsem1, ssem0, ssem1, accs):
    c = lax.axis_index("c")
    s = lax.axis_index("s")
    pltpu.sync_copy(zeros_ref, accs.at[pl.ds(s * RPT, RPT)])
    plsc.subcore_barrier()

    idx_r = (idx_r0, idx_r1, idx_r2, idx_r3)
    idx_c = (idx_c0, idx_c1, idx_c2, idx_c3)
    isem = (isem0, isem1, isem2, isem3)
    buf = (buf0, buf1)
    gsem = (gsem0, gsem1)
    ssem = (ssem0, ssem1)

    def start_idx(k, g):
        base = pl.multiple_of((s + NS * g) * CHUNK, 8)
        pltpu.async_copy(row_ref.at[pl.ds(base, CHUNK)], idx_r[k], isem[k])
        pltpu.async_copy(col_ref.at[pl.ds(base, CHUNK)], idx_c[k], isem[k])

    def wait_idx(k):
        pltpu.make_async_copy(row_ref.at[pl.ds(0, CHUNK)], idx_r[k], isem[k]).wait()
        pltpu.make_async_copy(col_ref.at[pl.ds(0, CHUNK)], idx_c[k], isem[k]).wait()

    def fire_gather(b, k):
        pltpu.async_copy(hp_ref.at[c].at[idx_r[k]], buf[b], gsem[b])

    def fire_scatter(b, k):
        pltpu.make_async_copy(hp_ref.at[c].at[idx_r[k]], buf[b], gsem[b]).wait()
        pltpu.async_copy(buf[b], accs.at[idx_c[k]], ssem[b], add=True)

    def drain_scatter(b, k):
        pltpu.make_async_copy(buf[b], accs.at[idx_c[k]], ssem[b]).wait()

    for k in range(4):
        start_idx(k, k)
    wait_idx(0)
    fire_gather(0, 0)
    wait_idx(1)
    fire_gather(1, 1)

    def body(j, carry):
        g0 = 4 * j
        for k in range(4):
            b = k % 2
            g = g0 + k
            fire_scatter(b, k)
            drain_scatter(b, k)

            @pl.when(g + 4 < FULL_PER_TILE)
            def _(k=k, g=g):
                start_idx(k, g + 4)

            @pl.when(g + 2 < FULL_PER_TILE)
            def _(b=b, k=k, g=g):
                wait_idx((k + 2) % 4)
                fire_gather(b, (k + 2) % 4)

        return carry

    lax.fori_loop(0, FULL_PER_TILE // 4, body, 0)

    @pl.when(s < TAIL_CHUNKS)
    def _():
        base = pl.multiple_of((E_CHUNKS - TAIL_CHUNKS + s) * CHUNK, 8)
        pltpu.sync_copy(row_ref.at[pl.ds(base, CHUNK)], idx_r[0])
        pltpu.sync_copy(col_ref.at[pl.ds(base, CHUNK)], idx_c[0])
        pltpu.sync_copy(hp_ref.at[c].at[idx_r[0]], buf[0])
        pltpu.sync_copy(buf[0], accs.at[idx_c[0]], add=True)

    plsc.subcore_barrier()
    pltpu.sync_copy(accs.at[pl.ds(s * RPT, RPT)], out_ref.at[c, pl.ds(s * RPT, RPT)])


_msg_kernel = functools.partial(
    pl.kernel,
    out_type=jax.ShapeDtypeStruct((NC, NP, HALF), jnp.float32),
    mesh=plsc.VectorSubcoreMesh(core_axis_name="c", subcore_axis_name="s"),
    scratch_types=[
        pltpu.VMEM((CHUNK,), jnp.int32),
        pltpu.VMEM((CHUNK,), jnp.int32),
        pltpu.VMEM((CHUNK,), jnp.int32),
        pltpu.VMEM((CHUNK,), jnp.int32),
        pltpu.VMEM((CHUNK,), jnp.int32),
        pltpu.VMEM((CHUNK,), jnp.int32),
        pltpu.VMEM((CHUNK,), jnp.int32),
        pltpu.VMEM((CHUNK,), jnp.int32),
        pltpu.VMEM((CHUNK, HALF), jnp.float32),
        pltpu.VMEM((CHUNK, HALF), jnp.float32),
        pltpu.SemaphoreType.DMA,
        pltpu.SemaphoreType.DMA,
        pltpu.SemaphoreType.DMA,
        pltpu.SemaphoreType.DMA,
        pltpu.SemaphoreType.DMA,
        pltpu.SemaphoreType.DMA,
        pltpu.SemaphoreType.DMA,
        pltpu.SemaphoreType.DMA,
        pltpu.VMEM_SHARED((NP, HALF), jnp.float32),
    ],
)(_msg_body)

BLK = 400
GRID = N // BLK


def _dinv_of(degp_ref):
    deg = degp_ref[0, :, 0] + degp_ref[1, :, 0] + 1.0
    return lax.rsqrt(deg)


def _l1_body(x_ref, w_ref, degp_ref, out_ref):
    dinv = _dinv_of(degp_ref)
    h = jnp.dot(x_ref[...], w_ref[...], preferred_element_type=jnp.float32)
    hp = h * dinv[:, None]
    out_ref[0] = hp[:, :HALF]
    out_ref[1] = hp[:, HALF:]


_l1 = pl.pallas_call(
    _l1_body,
    grid=(GRID,),
    in_specs=[
        pl.BlockSpec((BLK, IN_CH), lambda i: (i, 0)),
        pl.BlockSpec((IN_CH, HID), lambda i: (0, 0)),
        pl.BlockSpec((NC, BLK, 1), lambda i: (0, i, 0)),
    ],
    out_specs=pl.BlockSpec((NC, BLK, HALF), lambda i: (0, i, 0)),
    out_shape=jax.ShapeDtypeStruct((NC, NP, HALF), jnp.float32),
)


def _mid_body(acc_ref, hp_ref, degp_ref, w_ref, b_ref, out_ref):
    dinv = _dinv_of(degp_ref)
    t = jnp.concatenate([acc_ref[0] + hp_ref[0], acc_ref[1] + hp_ref[1]], axis=1)
    z = jnp.maximum(t * dinv[:, None] + b_ref[...], 0.0)
    h = jnp.dot(z, w_ref[...], preferred_element_type=jnp.float32)
    hp = h * dinv[:, None]
    out_ref[0] = hp[:, :HALF]
    out_ref[1] = hp[:, HALF:]


_mid = pl.pallas_call(
    _mid_body,
    grid=(GRID,),
    in_specs=[
        pl.BlockSpec((NC, BLK, HALF), lambda i: (0, i, 0)),
        pl.BlockSpec((NC, BLK, HALF), lambda i: (0, i, 0)),
        pl.BlockSpec((NC, BLK, 1), lambda i: (0, i, 0)),
        pl.BlockSpec((HID, HID), lambda i: (0, 0)),
        pl.BlockSpec((HID,), lambda i: (0,)),
    ],
    out_specs=pl.BlockSpec((NC, BLK, HALF), lambda i: (0, i, 0)),
    out_shape=jax.ShapeDtypeStruct((NC, NP, HALF), jnp.float32),
)


def _out_body(acc_ref, hp_ref, degp_ref, wc_ref, b_ref, bc_ref, out_ref):
    dinv = _dinv_of(degp_ref)
    t = jnp.concatenate([acc_ref[0] + hp_ref[0], acc_ref[1] + hp_ref[1]], axis=1)
    z = jnp.maximum(t * dinv[:, None] + b_ref[...], 0.0)
    out_ref[...] = jnp.dot(z, wc_ref[...], preferred_element_type=jnp.float32) + bc_ref[...]


_out = pl.pallas_call(
    _out_body,
    grid=(GRID,),
    in_specs=[
        pl.BlockSpec((NC, BLK, HALF), lambda i: (0, i, 0)),
        pl.BlockSpec((NC, BLK, HALF), lambda i: (0, i, 0)),
        pl.BlockSpec((NC, BLK, 1), lambda i: (0, i, 0)),
        pl.BlockSpec((HID, N_CLASSES), lambda i: (0, 0)),
        pl.BlockSpec((HID,), lambda i: (0,)),
        pl.BlockSpec((N_CLASSES,), lambda i: (0,)),
    ],
    out_specs=pl.BlockSpec((BLK, N_CLASSES), lambda i: (i, 0)),
    out_shape=jax.ShapeDtypeStruct((N, N_CLASSES), jnp.float32),
)


def kernel(x, edge_index, W1, b1, W2, b2, Wc, bc):
    row = edge_index[0].astype(jnp.int32)
    col = edge_index[1].astype(jnp.int32)
    ones16 = jnp.ones((CHUNK, DEG_W), jnp.float32)
    zeros16 = jnp.zeros((RPT, DEG_W), jnp.float32)
    zeros128 = jnp.zeros((RPT, HALF), jnp.float32)

    degp = _deg_kernel(col, ones16, zeros16)
    degp = degp[:, :, :1]
    hp1 = _l1(x, W1, degp)
    acc1 = _msg_kernel(hp1, row, col, zeros128)
    hp2 = _mid(acc1, hp1, degp, W2, b1)
    acc2 = _msg_kernel(hp2, row, col, zeros128)
    return _out(acc2, hp2, degp, Wc, b2, bc)


# speedup vs baseline: 1.2941x; 1.0007x over previous
import functools

import jax
import jax.numpy as jnp
from jax import lax
from jax.experimental import pallas as pl
from jax.experimental.pallas import tpu as pltpu
from jax.experimental.pallas import tpu_sc as plsc

N = 10000
E = 320000
IN_CH = 128
HID = 256
HALF = HID // 2
N_CLASSES = 2

NC = 2
NS = 16
NP = 10240
RPT = NP // NS

CHUNK = 128
E_CHUNKS = E // CHUNK
FULL_PER_TILE = E_CHUNKS // NS
TAIL_CHUNKS = E_CHUNKS - FULL_PER_TILE * NS

DEG_W = 128
DEG_OW = 128
NT = NC * NS
DEG_FULL = E_CHUNKS // NT
DEG_TAIL = E_CHUNKS - DEG_FULL * NT


def _deg_body(col_ref, ones_ref, zeros_ref, out_ref,
              idx0, idx1, idx2, idx3, idx4, idx5, idx6, idx7, ones_v,
              sem0, sem1, sem2, sem3, sem4, sem5, sem6, sem7,
              dsem0, dsem1, dsem2, dsem3, dsem4, dsem5, dsem6, dsem7, degs):
    c = lax.axis_index("c")
    s = lax.axis_index("s")
    tile = c * NS + s
    pltpu.sync_copy(zeros_ref, degs.at[pl.ds(s * RPT, RPT)])
    pltpu.sync_copy(ones_ref, ones_v)
    plsc.subcore_barrier()

    idx = (idx0, idx1, idx2, idx3, idx4, idx5, idx6, idx7)
    sem = (sem0, sem1, sem2, sem3, sem4, sem5, sem6, sem7)
    dsem = (dsem0, dsem1, dsem2, dsem3, dsem4, dsem5, dsem6, dsem7)

    def start_idx(k, g):
        base = pl.multiple_of((tile + NT * g) * CHUNK, 8)
        pltpu.async_copy(col_ref.at[pl.ds(base, CHUNK)], idx[k], sem[k])

    def scatter(k):
        pltpu.make_async_copy(col_ref.at[pl.ds(0, CHUNK)], idx[k], sem[k]).wait()
        pltpu.async_copy(ones_v, degs.at[idx[k]], dsem[k], add=True)

    def drain(k):
        pltpu.make_async_copy(ones_v, degs.at[idx[k]], dsem[k]).wait()

    for k in range(8):
        start_idx(k, k)

    def body(j, carry):
        g0 = 8 * j
        for k in range(8):
            g = g0 + k

            @pl.when(g < DEG_FULL)
            def _(k=k):
                scatter(k)

            @pl.when(jnp.logical_and(g >= 4, g - 4 < DEG_FULL))
            def _(k=k, g=g):
                drain((k + 4) % 8)

                @pl.when(g + 4 < DEG_FULL)
                def _(k=k, g=g):
                    start_idx((k + 4) % 8, g + 4)

        return carry

    lax.fori_loop(0, (DEG_FULL + 7) // 8 + 1, body, 0)

    @pl.when(tile < DEG_TAIL)
    def _():
        base = pl.multiple_of((E_CHUNKS - DEG_TAIL + tile) * CHUNK, 8)
        pltpu.sync_copy(col_ref.at[pl.ds(base, CHUNK)], idx[0])
        pltpu.sync_copy(ones_v, degs.at[idx[0]], add=True)

    plsc.subcore_barrier()
    pltpu.sync_copy(degs.at[pl.ds(s * RPT, RPT)], out_ref.at[c, pl.ds(s * RPT, RPT)])


_deg_kernel = functools.partial(
    pl.kernel,
    out_type=jax.ShapeDtypeStruct((NC, NP, DEG_OW), jnp.float32),
    mesh=plsc.VectorSubcoreMesh(core_axis_name="c", subcore_axis_name="s"),
    scratch_types=(
        [pltpu.VMEM((CHUNK,), jnp.int32)] * 8
        + [pltpu.VMEM((CHUNK, DEG_W), jnp.float32)]
        + [pltpu.SemaphoreType.DMA] * 16
        + [pltpu.VMEM_SHARED((NP, DEG_W), jnp.float32)]
    ),
)(_deg_body)


def _msg_body(hp_ref, row_ref, col_ref, zeros_ref, out_ref,
              idx_r0, idx_c0, idx_r1, idx_c1, idx_r2, idx_c2, idx_r3, idx_c3,
              buf0, buf1,
              isem0, isem1, isem2, isem3, gsem0, gsem1, ssem0, ssem1, accs):
    c = lax.axis_index("c")
    s = lax.axis_index("s")
    pltpu.sync_copy(zeros_ref, accs.at[pl.ds(s * RPT, RPT)])
    plsc.subcore_barrier()

    idx_r = (idx_r0, idx_r1, idx_r2, idx_r3)
    idx_c = (idx_c0, idx_c1, idx_c2, idx_c3)
    isem = (isem0, isem1, isem2, isem3)
    buf = (buf0, buf1)
    gsem = (gsem0, gsem1)
    ssem = (ssem0, ssem1)

    def start_idx(k, g):
        base = pl.multiple_of((s + NS * g) * CHUNK, 8)
        pltpu.async_copy(row_ref.at[pl.ds(base, CHUNK)], idx_r[k], isem[k])
        pltpu.async_copy(col_ref.at[pl.ds(base, CHUNK)], idx_c[k], isem[k])

    def wait_idx(k):
        pltpu.make_async_copy(row_ref.at[pl.ds(0, CHUNK)], idx_r[k], isem[k]).wait()
        pltpu.make_async_copy(col_ref.at[pl.ds(0, CHUNK)], idx_c[k], isem[k]).wait()

    def fire_gather(b, k):
        pltpu.async_copy(hp_ref.at[c].at[idx_r[k]], buf[b], gsem[b])

    def fire_scatter(b, k):
        pltpu.make_async_copy(hp_ref.at[c].at[idx_r[k]], buf[b], gsem[b]).wait()
        pltpu.async_copy(buf[b], accs.at[idx_c[k]], ssem[b], add=True)

    def drain_scatter(b, k):
        pltpu.make_async_copy(buf[b], accs.at[idx_c[k]], ssem[b]).wait()

    for k in range(4):
        start_idx(k, k)
    wait_idx(0)
    fire_gather(0, 0)
    wait_idx(1)
    fire_gather(1, 1)

    def body(j, carry):
        g0 = 4 * j
        for k in range(4):
            b = k % 2
            g = g0 + k
            fire_scatter(b, k)
            drain_scatter(b, k)

            @pl.when(g + 4 < FULL_PER_TILE)
            def _(k=k, g=g):
                start_idx(k, g + 4)

            @pl.when(g + 2 < FULL_PER_TILE)
            def _(b=b, k=k, g=g):
                wait_idx((k + 2) % 4)
                fire_gather(b, (k + 2) % 4)

        return carry

    lax.fori_loop(0, FULL_PER_TILE // 4, body, 0)

    @pl.when(s < TAIL_CHUNKS)
    def _():
        base = pl.multiple_of((E_CHUNKS - TAIL_CHUNKS + s) * CHUNK, 8)
        pltpu.sync_copy(row_ref.at[pl.ds(base, CHUNK)], idx_r[0])
        pltpu.sync_copy(col_ref.at[pl.ds(base, CHUNK)], idx_c[0])
        pltpu.sync_copy(hp_ref.at[c].at[idx_r[0]], buf[0])
        pltpu.sync_copy(buf[0], accs.at[idx_c[0]], add=True)

    plsc.subcore_barrier()
    pltpu.sync_copy(accs.at[pl.ds(s * RPT, RPT)], out_ref.at[c, pl.ds(s * RPT, RPT)])


_msg_kernel = functools.partial(
    pl.kernel,
    out_type=jax.ShapeDtypeStruct((NC, NP, HALF), jnp.float32),
    mesh=plsc.VectorSubcoreMesh(core_axis_name="c", subcore_axis_name="s"),
    scratch_types=[
        pltpu.VMEM((CHUNK,), jnp.int32),
        pltpu.VMEM((CHUNK,), jnp.int32),
        pltpu.VMEM((CHUNK,), jnp.int32),
        pltpu.VMEM((CHUNK,), jnp.int32),
        pltpu.VMEM((CHUNK,), jnp.int32),
        pltpu.VMEM((CHUNK,), jnp.int32),
        pltpu.VMEM((CHUNK,), jnp.int32),
        pltpu.VMEM((CHUNK,), jnp.int32),
        pltpu.VMEM((CHUNK, HALF), jnp.float32),
        pltpu.VMEM((CHUNK, HALF), jnp.float32),
        pltpu.SemaphoreType.DMA,
        pltpu.SemaphoreType.DMA,
        pltpu.SemaphoreType.DMA,
        pltpu.SemaphoreType.DMA,
        pltpu.SemaphoreType.DMA,
        pltpu.SemaphoreType.DMA,
        pltpu.SemaphoreType.DMA,
        pltpu.SemaphoreType.DMA,
        pltpu.VMEM_SHARED((NP, HALF), jnp.float32),
    ],
)(_msg_body)

BLK = 400
GRID = N // BLK


def _dinv_of(degp_ref):
    deg = degp_ref[0, :, 0] + degp_ref[1, :, 0] + 1.0
    return lax.rsqrt(deg)


def _l1_body(x_ref, w_ref, degp_ref, out_ref):
    dinv = _dinv_of(degp_ref)
    h = jnp.dot(x_ref[...], w_ref[...], preferred_element_type=jnp.float32)
    hp = h * dinv[:, None]
    out_ref[0] = hp[:, :HALF]
    out_ref[1] = hp[:, HALF:]


_l1 = pl.pallas_call(
    _l1_body,
    grid=(GRID,),
    in_specs=[
        pl.BlockSpec((BLK, IN_CH), lambda i: (i, 0)),
        pl.BlockSpec((IN_CH, HID), lambda i: (0, 0)),
        pl.BlockSpec((NC, BLK, 1), lambda i: (0, i, 0)),
    ],
    out_specs=pl.BlockSpec((NC, BLK, HALF), lambda i: (0, i, 0)),
    out_shape=jax.ShapeDtypeStruct((NC, NP, HALF), jnp.float32),
)


def _mid_body(acc_ref, hp_ref, degp_ref, w_ref, b_ref, out_ref):
    dinv = _dinv_of(degp_ref)
    t = jnp.concatenate([acc_ref[0] + hp_ref[0], acc_ref[1] + hp_ref[1]], axis=1)
    z = jnp.maximum(t * dinv[:, None] + b_ref[...], 0.0)
    h = jnp.dot(z, w_ref[...], preferred_element_type=jnp.float32)
    hp = h * dinv[:, None]
    out_ref[0] = hp[:, :HALF]
    out_ref[1] = hp[:, HALF:]


_mid = pl.pallas_call(
    _mid_body,
    grid=(GRID,),
    in_specs=[
        pl.BlockSpec((NC, BLK, HALF), lambda i: (0, i, 0)),
        pl.BlockSpec((NC, BLK, HALF), lambda i: (0, i, 0)),
        pl.BlockSpec((NC, BLK, 1), lambda i: (0, i, 0)),
        pl.BlockSpec((HID, HID), lambda i: (0, 0)),
        pl.BlockSpec((HID,), lambda i: (0,)),
    ],
    out_specs=pl.BlockSpec((NC, BLK, HALF), lambda i: (0, i, 0)),
    out_shape=jax.ShapeDtypeStruct((NC, NP, HALF), jnp.float32),
)


def _out_body(acc_ref, hp_ref, degp_ref, wc_ref, b_ref, bc_ref, out_ref):
    dinv = _dinv_of(degp_ref)
    t = jnp.concatenate([acc_ref[0] + hp_ref[0], acc_ref[1] + hp_ref[1]], axis=1)
    z = jnp.maximum(t * dinv[:, None] + b_ref[...], 0.0)
    out_ref[...] = jnp.dot(z, wc_ref[...], preferred_element_type=jnp.float32) + bc_ref[...]


_out = pl.pallas_call(
    _out_body,
    grid=(GRID,),
    in_specs=[
        pl.BlockSpec((NC, BLK, HALF), lambda i: (0, i, 0)),
        pl.BlockSpec((NC, BLK, HALF), lambda i: (0, i, 0)),
        pl.BlockSpec((NC, BLK, 1), lambda i: (0, i, 0)),
        pl.BlockSpec((HID, N_CLASSES), lambda i: (0, 0)),
        pl.BlockSpec((HID,), lambda i: (0,)),
        pl.BlockSpec((N_CLASSES,), lambda i: (0,)),
    ],
    out_specs=pl.BlockSpec((BLK, N_CLASSES), lambda i: (i, 0)),
    out_shape=jax.ShapeDtypeStruct((N, N_CLASSES), jnp.float32),
)


def kernel(x, edge_index, W1, b1, W2, b2, Wc, bc):
    row = edge_index[0].astype(jnp.int32)
    col = edge_index[1].astype(jnp.int32)
    ones16 = jnp.ones((CHUNK, DEG_W), jnp.float32)
    zeros16 = jnp.zeros((RPT, DEG_W), jnp.float32)
    zeros128 = jnp.zeros((RPT, HALF), jnp.float32)

    degp = _deg_kernel(col, ones16, zeros16)
    degp = degp[:, :, :1]
    hp1 = _l1(x, W1, degp)
    acc1 = _msg_kernel(hp1, row, col, zeros128)
    hp2 = _mid(acc1, hp1, degp, W2, b1)
    acc2 = _msg_kernel(hp2, row, col, zeros128)
    return _out(acc2, hp2, degp, Wc, b2, bc)


# speedup vs baseline: 1.3742x; 1.0620x over previous
import functools

import jax
import jax.numpy as jnp
from jax import lax
from jax.experimental import pallas as pl
from jax.experimental.pallas import tpu as pltpu
from jax.experimental.pallas import tpu_sc as plsc

N = 10000
E = 320000
IN_CH = 128
HID = 256
HALF = HID // 2
N_CLASSES = 2

NC = 2
NS = 16
NP = 10240
RPT = NP // NS

CHUNK = 80
E_CHUNKS = E // CHUNK
FULL_PER_TILE = E_CHUNKS // NS
TAIL_CHUNKS = E_CHUNKS - FULL_PER_TILE * NS

DEG_W = 128
DEG_OW = 128
NT = NC * NS
DEG_FULL = E_CHUNKS // NT
DEG_TAIL = E_CHUNKS - DEG_FULL * NT


def _deg_body(col_ref, ones_ref, zeros_ref, out_ref,
              idx0, idx1, idx2, idx3, idx4, idx5, idx6, idx7, ones_v,
              sem0, sem1, sem2, sem3, sem4, sem5, sem6, sem7,
              dsem0, dsem1, dsem2, dsem3, dsem4, dsem5, dsem6, dsem7, degs):
    c = lax.axis_index("c")
    s = lax.axis_index("s")
    tile = c * NS + s
    pltpu.sync_copy(zeros_ref, degs.at[pl.ds(s * RPT, RPT)])
    pltpu.sync_copy(ones_ref, ones_v)
    plsc.subcore_barrier()

    idx = (idx0, idx1, idx2, idx3, idx4, idx5, idx6, idx7)
    sem = (sem0, sem1, sem2, sem3, sem4, sem5, sem6, sem7)
    dsem = (dsem0, dsem1, dsem2, dsem3, dsem4, dsem5, dsem6, dsem7)

    def start_idx(k, g):
        base = pl.multiple_of((tile + NT * g) * CHUNK, 8)
        pltpu.async_copy(col_ref.at[pl.ds(base, CHUNK)], idx[k], sem[k])

    def scatter(k):
        pltpu.make_async_copy(col_ref.at[pl.ds(0, CHUNK)], idx[k], sem[k]).wait()
        pltpu.async_copy(ones_v, degs.at[idx[k]], dsem[k], add=True)

    def drain(k):
        pltpu.make_async_copy(ones_v, degs.at[idx[k]], dsem[k]).wait()

    for k in range(8):
        start_idx(k, k)

    def body(j, carry):
        g0 = 8 * j
        for k in range(8):
            g = g0 + k

            @pl.when(g < DEG_FULL)
            def _(k=k):
                scatter(k)

            @pl.when(jnp.logical_and(g >= 4, g - 4 < DEG_FULL))
            def _(k=k, g=g):
                drain((k + 4) % 8)

                @pl.when(g + 4 < DEG_FULL)
                def _(k=k, g=g):
                    start_idx((k + 4) % 8, g + 4)

        return carry

    lax.fori_loop(0, (DEG_FULL + 7) // 8 + 1, body, 0)

    @pl.when(tile < DEG_TAIL)
    def _():
        base = pl.multiple_of((E_CHUNKS - DEG_TAIL + tile) * CHUNK, 8)
        pltpu.sync_copy(col_ref.at[pl.ds(base, CHUNK)], idx[0])
        pltpu.sync_copy(ones_v, degs.at[idx[0]], add=True)

    plsc.subcore_barrier()
    pltpu.sync_copy(degs.at[pl.ds(s * RPT, RPT)], out_ref.at[c, pl.ds(s * RPT, RPT)])


_deg_kernel = functools.partial(
    pl.kernel,
    out_type=jax.ShapeDtypeStruct((NC, NP, DEG_OW), jnp.float32),
    mesh=plsc.VectorSubcoreMesh(core_axis_name="c", subcore_axis_name="s"),
    scratch_types=(
        [pltpu.VMEM((CHUNK,), jnp.int32)] * 8
        + [pltpu.VMEM((CHUNK, DEG_W), jnp.float32)]
        + [pltpu.SemaphoreType.DMA] * 16
        + [pltpu.VMEM_SHARED((NP, DEG_W), jnp.float32)]
    ),
)(_deg_body)


def _msg_body(hp_ref, row_ref, col_ref, zeros_ref, out_ref,
              idx_r0, idx_c0, idx_r1, idx_c1, idx_r2, idx_c2, idx_r3, idx_c3,
              idx_r4, idx_c4, idx_r5, idx_c5,
              buf0, buf1, buf2,
              isem0, isem1, isem2, isem3, isem4, isem5,
              gsem0, gsem1, gsem2, ssem0, ssem1, ssem2, accs):
    c = lax.axis_index("c")
    s = lax.axis_index("s")
    pltpu.sync_copy(zeros_ref, accs.at[pl.ds(s * RPT, RPT)])
    plsc.subcore_barrier()

    idx_r = (idx_r0, idx_r1, idx_r2, idx_r3, idx_r4, idx_r5)
    idx_c = (idx_c0, idx_c1, idx_c2, idx_c3, idx_c4, idx_c5)
    isem = (isem0, isem1, isem2, isem3, isem4, isem5)
    buf = (buf0, buf1, buf2)
    gsem = (gsem0, gsem1, gsem2)
    ssem = (ssem0, ssem1, ssem2)

    def start_idx(k, g):
        base = pl.multiple_of((s + NS * g) * CHUNK, 8)
        pltpu.async_copy(row_ref.at[pl.ds(base, CHUNK)], idx_r[k], isem[k])
        pltpu.async_copy(col_ref.at[pl.ds(base, CHUNK)], idx_c[k], isem[k])

    def wait_idx(k):
        pltpu.make_async_copy(row_ref.at[pl.ds(0, CHUNK)], idx_r[k], isem[k]).wait()
        pltpu.make_async_copy(col_ref.at[pl.ds(0, CHUNK)], idx_c[k], isem[k]).wait()

    def fire_gather(b, k):
        pltpu.async_copy(hp_ref.at[c].at[idx_r[k]], buf[b], gsem[b])

    def fire_scatter(b, k):
        pltpu.make_async_copy(hp_ref.at[c].at[idx_r[k]], buf[b], gsem[b]).wait()
        pltpu.async_copy(buf[b], accs.at[idx_c[k]], ssem[b], add=True)

    def drain_scatter(b, k):
        pltpu.make_async_copy(buf[b], accs.at[idx_c[k]], ssem[b]).wait()

    for k in range(6):
        start_idx(k, k)
    wait_idx(0)
    fire_gather(0, 0)
    wait_idx(1)
    fire_gather(1, 1)

    def body(j, carry):
        g0 = 6 * j
        for k in range(6):
            b = k % 3
            g = g0 + k

            @pl.when(g < FULL_PER_TILE)
            def _(b=b, k=k):
                fire_scatter(b, k)

            @pl.when(jnp.logical_and(g >= 1, g - 1 < FULL_PER_TILE))
            def _(g=g, k=k):
                drain_scatter((k + 2) % 3, (k + 5) % 6)

                @pl.when(g + 5 < FULL_PER_TILE)
                def _(g=g, k=k):
                    start_idx((k + 5) % 6, g + 5)

            @pl.when(g + 2 < FULL_PER_TILE)
            def _(g=g, k=k):
                wait_idx((k + 2) % 6)
                fire_gather((k + 2) % 3, (k + 2) % 6)

        return carry

    lax.fori_loop(0, FULL_PER_TILE // 6 + 1, body, 0)

    @pl.when(s < TAIL_CHUNKS)
    def _():
        base = pl.multiple_of((E_CHUNKS - TAIL_CHUNKS + s) * CHUNK, 8)
        pltpu.sync_copy(row_ref.at[pl.ds(base, CHUNK)], idx_r[0])
        pltpu.sync_copy(col_ref.at[pl.ds(base, CHUNK)], idx_c[0])
        pltpu.sync_copy(hp_ref.at[c].at[idx_r[0]], buf[0])
        pltpu.sync_copy(buf[0], accs.at[idx_c[0]], add=True)

    plsc.subcore_barrier()
    pltpu.sync_copy(accs.at[pl.ds(s * RPT, RPT)], out_ref.at[c, pl.ds(s * RPT, RPT)])


_msg_kernel = functools.partial(
    pl.kernel,
    out_type=jax.ShapeDtypeStruct((NC, NP, HALF), jnp.float32),
    mesh=plsc.VectorSubcoreMesh(core_axis_name="c", subcore_axis_name="s"),
    scratch_types=(
        [pltpu.VMEM((CHUNK,), jnp.int32)] * 12
        + [pltpu.VMEM((CHUNK, HALF), jnp.float32)] * 3
        + [pltpu.SemaphoreType.DMA] * 12
        + [pltpu.VMEM_SHARED((NP, HALF), jnp.float32)]
    ),
)(_msg_body)

BLK = 400
GRID = N // BLK


def _dinv_of(degp_ref):
    deg = degp_ref[0, :, 0] + degp_ref[1, :, 0] + 1.0
    return lax.rsqrt(deg)


def _l1_body(x_ref, w_ref, degp_ref, out_ref):
    dinv = _dinv_of(degp_ref)
    h = jnp.dot(x_ref[...], w_ref[...], preferred_element_type=jnp.float32)
    hp = h * dinv[:, None]
    out_ref[0] = hp[:, :HALF]
    out_ref[1] = hp[:, HALF:]


_l1 = pl.pallas_call(
    _l1_body,
    grid=(GRID,),
    in_specs=[
        pl.BlockSpec((BLK, IN_CH), lambda i: (i, 0)),
        pl.BlockSpec((IN_CH, HID), lambda i: (0, 0)),
        pl.BlockSpec((NC, BLK, 1), lambda i: (0, i, 0)),
    ],
    out_specs=pl.BlockSpec((NC, BLK, HALF), lambda i: (0, i, 0)),
    out_shape=jax.ShapeDtypeStruct((NC, NP, HALF), jnp.float32),
)


def _mid_body(acc_ref, hp_ref, degp_ref, w_ref, b_ref, out_ref):
    dinv = _dinv_of(degp_ref)
    t = jnp.concatenate([acc_ref[0] + hp_ref[0], acc_ref[1] + hp_ref[1]], axis=1)
    z = jnp.maximum(t * dinv[:, None] + b_ref[...], 0.0)
    h = jnp.dot(z, w_ref[...], preferred_element_type=jnp.float32)
    hp = h * dinv[:, None]
    out_ref[0] = hp[:, :HALF]
    out_ref[1] = hp[:, HALF:]


_mid = pl.pallas_call(
    _mid_body,
    grid=(GRID,),
    in_specs=[
        pl.BlockSpec((NC, BLK, HALF), lambda i: (0, i, 0)),
        pl.BlockSpec((NC, BLK, HALF), lambda i: (0, i, 0)),
        pl.BlockSpec((NC, BLK, 1), lambda i: (0, i, 0)),
        pl.BlockSpec((HID, HID), lambda i: (0, 0)),
        pl.BlockSpec((HID,), lambda i: (0,)),
    ],
    out_specs=pl.BlockSpec((NC, BLK, HALF), lambda i: (0, i, 0)),
    out_shape=jax.ShapeDtypeStruct((NC, NP, HALF), jnp.float32),
)


def _out_body(acc_ref, hp_ref, degp_ref, wc_ref, b_ref, bc_ref, out_ref):
    dinv = _dinv_of(degp_ref)
    t = jnp.concatenate([acc_ref[0] + hp_ref[0], acc_ref[1] + hp_ref[1]], axis=1)
    z = jnp.maximum(t * dinv[:, None] + b_ref[...], 0.0)
    out_ref[...] = jnp.dot(z, wc_ref[...], preferred_element_type=jnp.float32) + bc_ref[...]


_out = pl.pallas_call(
    _out_body,
    grid=(GRID,),
    in_specs=[
        pl.BlockSpec((NC, BLK, HALF), lambda i: (0, i, 0)),
        pl.BlockSpec((NC, BLK, HALF), lambda i: (0, i, 0)),
        pl.BlockSpec((NC, BLK, 1), lambda i: (0, i, 0)),
        pl.BlockSpec((HID, N_CLASSES), lambda i: (0, 0)),
        pl.BlockSpec((HID,), lambda i: (0,)),
        pl.BlockSpec((N_CLASSES,), lambda i: (0,)),
    ],
    out_specs=pl.BlockSpec((BLK, N_CLASSES), lambda i: (i, 0)),
    out_shape=jax.ShapeDtypeStruct((N, N_CLASSES), jnp.float32),
)


def kernel(x, edge_index, W1, b1, W2, b2, Wc, bc):
    row = edge_index[0].astype(jnp.int32)
    col = edge_index[1].astype(jnp.int32)
    ones16 = jnp.ones((CHUNK, DEG_W), jnp.float32)
    zeros16 = jnp.zeros((RPT, DEG_W), jnp.float32)
    zeros128 = jnp.zeros((RPT, HALF), jnp.float32)

    degp = _deg_kernel(col, ones16, zeros16)
    degp = degp[:, :, :1]
    hp1 = _l1(x, W1, degp)
    acc1 = _msg_kernel(hp1, row, col, zeros128)
    hp2 = _mid(acc1, hp1, degp, W2, b1)
    acc2 = _msg_kernel(hp2, row, col, zeros128)
    return _out(acc2, hp2, degp, Wc, b2, bc)
